# Initial kernel scaffold; baseline (speedup 1.0000x reference)
import jax
import jax.numpy as jnp
from jax.experimental import pallas as pl


def kernel(x, edge_index, W1, b1, W2, b2):
    raise NotImplementedError("write your pallas kernel here")


# speedup vs baseline: 20.0595x; 20.0595x over previous
import functools

import jax
import jax.numpy as jnp
from jax import lax
from jax.experimental import pallas as pl
from jax.experimental.pallas import tpu as pltpu
from jax.experimental.pallas import tpu_sc as plsc

N = 10000
D = 128
E = 320000
NC = 2
NS = 16
NW = NC * NS
CHUNK = 128
KCH = 79
EPAD = NW * KCH * CHUNK
NPAD = 10240
DUMMY = N
RPT = NPAD // NS

_mesh = plsc.VectorSubcoreMesh(core_axis_name="c", subcore_axis_name="s")


def _sc_degree_body(cols_hbm, onehot_hbm, zeros_hbm, out_hbm, cols_v, src_v, zb_v, acc_sh):
    c = lax.axis_index("c")
    s = lax.axis_index("s")
    wid = s * NC + c
    pltpu.sync_copy(onehot_hbm, src_v)
    pltpu.sync_copy(zeros_hbm, zb_v)
    for m in range(RPT // CHUNK):
        pltpu.sync_copy(zb_v, acc_sh.at[pl.ds(s * RPT + m * CHUNK, CHUNK)])
    plsc.subcore_barrier()

    pltpu.sync_copy(cols_hbm.at[wid], cols_v)

    def hist_body(j, carry):
        pltpu.sync_copy(src_v, acc_sh.at[cols_v.at[j]], add=True)
        return carry

    lax.fori_loop(0, KCH, hist_body, 0)
    plsc.subcore_barrier()
    pltpu.sync_copy(acc_sh.at[pl.ds(s * RPT, RPT)],
                    out_hbm.at[c, pl.ds(s * RPT, RPT)])


def _make_sc_degree(interpret=False):
    return pl.kernel(
        _sc_degree_body,
        out_type=jax.ShapeDtypeStruct((NC, NPAD, D), jnp.float32),
        mesh=_mesh,
        scratch_types=[
            pltpu.VMEM((KCH, CHUNK), jnp.int32),
            pltpu.VMEM((CHUNK, D), jnp.float32),
            pltpu.VMEM((CHUNK, D), jnp.float32),
            pltpu.VMEM_SHARED((NPAD, D), jnp.float32),
        ],
        interpret=interpret,
    )


_sc_degree = _make_sc_degree()


def _sc_agg_body(u_hbm, rows_hbm, cols_hbm, out_hbm, rows_v, cols_v, msg_v, acc_sh, sem):
    c = lax.axis_index("c")
    s = lax.axis_index("s")
    wid = s * NC + c
    zeros16 = jnp.zeros((16,), jnp.float32)

    def zero_body(i, carry):
        for k in range(D // 16):
            msg_v[i, pl.ds(k * 16, 16)] = zeros16
        return carry

    lax.fori_loop(0, CHUNK, zero_body, 0)
    for m in range(RPT // CHUNK):
        pltpu.sync_copy(msg_v, acc_sh.at[pl.ds(s * RPT + m * CHUNK, CHUNK)])
    plsc.subcore_barrier()

    pltpu.sync_copy(rows_hbm.at[wid], rows_v)
    pltpu.sync_copy(cols_hbm.at[wid], cols_v)

    def edge_body(j, carry):
        pltpu.async_copy(u_hbm.at[rows_v.at[j]], msg_v, sem).wait()
        pltpu.sync_copy(msg_v, acc_sh.at[cols_v.at[j]], add=True)
        return carry

    lax.fori_loop(0, KCH, edge_body, 0)
    plsc.subcore_barrier()
    pltpu.sync_copy(acc_sh.at[pl.ds(s * RPT, RPT)],
                    out_hbm.at[c, pl.ds(s * RPT, RPT)])


def _make_sc_agg(interpret=False):
    return pl.kernel(
        _sc_agg_body,
        out_type=jax.ShapeDtypeStruct((NC, NPAD, D), jnp.float32),
        mesh=_mesh,
        scratch_types=[
            pltpu.VMEM((KCH, CHUNK), jnp.int32),
            pltpu.VMEM((KCH, CHUNK), jnp.int32),
            pltpu.VMEM((CHUNK, D), jnp.float32),
            pltpu.VMEM_SHARED((NPAD, D), jnp.float32),
            pltpu.SemaphoreType.DMA,
        ],
        interpret=interpret,
    )


_sc_agg = _make_sc_agg()


def _prescale_body(x_ref, d0_ref, d1_ref, u_ref, dinv_ref):
    deg = d0_ref[...][:, 0:1] + d1_ref[...][:, 0:1] + 1.0
    dinv = lax.rsqrt(deg)
    dinv_ref[...] = dinv
    u_ref[...] = x_ref[...] * dinv


_prescale = pl.pallas_call(
    _prescale_body,
    out_shape=[
        jax.ShapeDtypeStruct((NPAD, D), jnp.float32),
        jax.ShapeDtypeStruct((NPAD, 1), jnp.float32),
    ],
)

BLK = 1024


def _final_body(p0, p1, x_ref, dv_ref, w1, b1r, w2, b2r, o3, o1, o2):
    dinv = dv_ref[...]
    y = dinv * (p0[...] + p1[...]) + (dinv * dinv) * x_ref[...]
    h1 = jnp.maximum(
        jnp.dot(y, w1[...], preferred_element_type=jnp.float32) + b1r[...], 0.0)
    h2 = jnp.maximum(
        jnp.dot(y, w2[...], preferred_element_type=jnp.float32) + b2r[...], 0.0)
    o1[...] = h1
    o2[...] = h2
    o3[...] = h1 + h2


_final = pl.pallas_call(
    _final_body,
    grid=(NPAD // BLK,),
    in_specs=[
        pl.BlockSpec((BLK, D), lambda i: (i, 0)),
        pl.BlockSpec((BLK, D), lambda i: (i, 0)),
        pl.BlockSpec((BLK, D), lambda i: (i, 0)),
        pl.BlockSpec((BLK, 1), lambda i: (i, 0)),
        pl.BlockSpec((D, D), lambda i: (0, 0)),
        pl.BlockSpec((1, D), lambda i: (0, 0)),
        pl.BlockSpec((D, D), lambda i: (0, 0)),
        pl.BlockSpec((1, D), lambda i: (0, 0)),
    ],
    out_specs=[pl.BlockSpec((BLK, D), lambda i: (i, 0))] * 3,
    out_shape=[jax.ShapeDtypeStruct((NPAD, D), jnp.float32)] * 3,
)


def kernel(x, edge_index, W1, b1, W2, b2):
    ei = edge_index.astype(jnp.int32)
    rows = jnp.concatenate(
        [ei[0], jnp.zeros((EPAD - E,), jnp.int32)]).reshape(NW, KCH, CHUNK)
    cols = jnp.concatenate(
        [ei[1], jnp.full((EPAD - E,), DUMMY, jnp.int32)]).reshape(NW, KCH, CHUNK)
    xp = jnp.pad(x, ((0, NPAD - N), (0, 0)))

    onehot = jnp.zeros((CHUNK, D), jnp.float32).at[:, 0].set(1.0)
    zeros2d = jnp.zeros((CHUNK, D), jnp.float32)
    degp = _sc_degree(cols, onehot, zeros2d)
    u, dinv = _prescale(xp, degp[0], degp[1])
    segsum = _sc_agg(u, rows, cols)
    x3, x1, x2 = _final(segsum[0], segsum[1], xp, dinv,
                        W1, b1.reshape(1, D), W2, b2.reshape(1, D))
    return (x3[:N], x1[:N], x2[:N])
